# baseline (speedup 1.0000x reference)
import math

import jax
import jax.numpy as jnp
from jax.experimental import pallas as pl
from jax.experimental.pallas import tpu as pltpu

_MAX_REL = 128
_SQ = 2048
_SK = 2048
_TQ = 256
_TK = 512
_L = ((_TQ + _TK - 1 + 127) // 128) * 128
_G = math.gcd(_TQ, _TK)
_ND = (_SQ - _TQ + _SK - _TK) // _G + 1
_HEAD = _SQ - _MAX_REL - 1


def _add_bias_body(f_ref, x_ref, o_ref):
    f = f_ref[0, 0, :]
    fb = jnp.broadcast_to(f[None, :], (_TQ, _L))
    bias = pltpu.roll(fb, _L - _TQ + 1, axis=1, stride=1, stride_axis=0)
    o_ref[...] = x_ref[...] + bias[None, :, :_TK]


def kernel(inputs, relative_biases):
    t = relative_biases
    tail = _G * (_ND - 1) + _L - _HEAD - (2 * _MAX_REL + 1)
    e = jnp.concatenate(
        [jnp.full((_HEAD,), t[0], t.dtype), t, jnp.full((tail,), t[256], t.dtype)]
    )
    f_all = jnp.stack([e[_G * d : _G * d + _L] for d in range(_ND)])
    f_all = f_all.reshape(_ND, 1, _L)

    b = inputs.shape[0]
    grid = (_SQ // _TQ, _SK // _TK)

    def f_idx(qi, ki):
        return ((ki * _TK - qi * _TQ + _SQ - _TQ) // _G, 0, 0)

    return pl.pallas_call(
        _add_bias_body,
        grid=grid,
        in_specs=[
            pl.BlockSpec((1, 1, _L), f_idx),
            pl.BlockSpec((b, _TQ, _TK), lambda qi, ki: (0, qi, ki)),
        ],
        out_specs=pl.BlockSpec((b, _TQ, _TK), lambda qi, ki: (0, qi, ki)),
        out_shape=jax.ShapeDtypeStruct(inputs.shape, inputs.dtype),
    )(f_all, inputs)
